# baseline (speedup 1.0000x reference)
import functools

import jax
import jax.numpy as jnp
from jax import lax
from jax.experimental import pallas as pl
from jax.experimental.pallas import tpu as pltpu
from jax.experimental.pallas import tpu_sc as plsc

_BATCH = 16384
_HIST = 20
_D = 64
_B = _BATCH * _HIST
_NC = 2
_NS = 16
_NW = _NC * _NS
_CHUNK = 128
_PERW = _B // _NW
_CH = _PERW // _CHUNK


def _make_gather():
    mesh = plsc.VectorSubcoreMesh(core_axis_name="c", subcore_axis_name="s")

    @functools.partial(
        pl.kernel,
        mesh=mesh,
        compiler_params=pltpu.CompilerParams(use_tc_tiling_on_sc=False),
        out_type=jax.ShapeDtypeStruct((_B, _D), jnp.float32),
        scratch_types=[
            pltpu.VMEM((_CH, _CHUNK), jnp.int32),
            pltpu.VMEM((_CHUNK, _D), jnp.float32),
            pltpu.SemaphoreType.DMA,
        ],
    )
    def gather(idx_hbm, table_hbm, out_hbm, idx_v, rows_v, sem):
        wid = lax.axis_index("s") * _NC + lax.axis_index("c")
        base = wid * _PERW
        pltpu.sync_copy(idx_hbm.at[wid], idx_v)

        def body(g, carry):
            pltpu.async_copy(table_hbm.at[idx_v.at[g]], rows_v, sem).wait()
            pltpu.sync_copy(rows_v, out_hbm.at[pl.ds(base + g * _CHUNK, _CHUNK)])
            return carry

        lax.fori_loop(0, _CH, body, 0)

    return gather


_gather = _make_gather()


def kernel(x, table):
    idx = x.reshape(_NW, _CH, _CHUNK).astype(jnp.int32)
    out = _gather(idx, table)
    return out.reshape(_BATCH, _HIST, _D)


# speedup vs baseline: 1.0654x; 1.0654x over previous
import functools

import jax
import jax.numpy as jnp
from jax import lax
from jax.experimental import pallas as pl
from jax.experimental.pallas import tpu as pltpu
from jax.experimental.pallas import tpu_sc as plsc

_BATCH = 16384
_HIST = 20
_D = 64
_B = _BATCH * _HIST
_NC = 2
_NS = 16
_NW = _NC * _NS
_CHUNK = 128
_PERW = _B // _NW
_CH = _PERW // _CHUNK
_K = 4
_SUPER = _K * _CHUNK
_S = _CH // _K
_NBUF = 2


def _make_gather():
    mesh = plsc.VectorSubcoreMesh(core_axis_name="c", subcore_axis_name="s")

    @functools.partial(
        pl.kernel,
        mesh=mesh,
        compiler_params=pltpu.CompilerParams(use_tc_tiling_on_sc=False),
        out_type=jax.ShapeDtypeStruct((_B, _D), jnp.float32),
        scratch_types=[
            pltpu.VMEM((_CH, _CHUNK), jnp.int32),
            pltpu.VMEM((_NBUF, _SUPER, _D), jnp.float32),
            pltpu.SemaphoreType.DMA,
            pltpu.SemaphoreType.DMA,
            pltpu.SemaphoreType.DMA,
            pltpu.SemaphoreType.DMA,
        ],
    )
    def gather(idx_hbm, table_hbm, out_hbm, idx_v, rows, gs0, gs1, ss0, ss1):
        gsem = (gs0, gs1)
        ssem = (ss0, ss1)
        wid = lax.axis_index("s") * _NC + lax.axis_index("c")
        base = wid * _PERW
        pltpu.sync_copy(idx_hbm.at[wid], idx_v)

        def fire(s, b):
            for j in range(_K):
                pltpu.async_copy(
                    table_hbm.at[idx_v.at[s * _K + j]],
                    rows.at[b, pl.ds(j * _CHUNK, _CHUNK)],
                    gsem[b],
                )

        def wait_gather(b):
            pltpu.make_async_copy(
                table_hbm.at[pl.ds(0, _SUPER)], rows.at[b], gsem[b]
            ).wait()

        def store(s, b):
            pltpu.async_copy(
                rows.at[b], out_hbm.at[pl.ds(base + s * _SUPER, _SUPER)], ssem[b]
            )

        def wait_store(b):
            pltpu.make_async_copy(
                table_hbm.at[pl.ds(0, _SUPER)], rows.at[b], ssem[b]
            ).wait()

        for b in range(_NBUF):
            fire(b, b)

        def body(i, carry):
            s0 = i * _NBUF
            for b in range(_NBUF):
                s = s0 + b
                wait_gather(b)
                store(s, b)
                wait_store(b)
                fire(s + _NBUF, b)
            return carry

        lax.fori_loop(0, (_S - _NBUF) // _NBUF, body, 0)

        for b in range(_NBUF):
            s = _S - _NBUF + b
            wait_gather(b)
            store(s, b)
            wait_store(b)

    return gather


_gather = _make_gather()


def kernel(x, table):
    idx = x.reshape(_NW, _CH, _CHUNK).astype(jnp.int32)
    out = _gather(idx, table)
    return out.reshape(_BATCH, _HIST, _D)
